# Initial kernel scaffold; baseline (speedup 1.0000x reference)
import jax
import jax.numpy as jnp
from jax.experimental import pallas as pl


def kernel(z, edge_index):
    raise NotImplementedError("write your pallas kernel here")


# speedup vs baseline: 1.0962x; 1.0962x over previous
import functools

import jax
import jax.numpy as jnp
from jax import lax
from jax.experimental import pallas as pl
from jax.experimental.pallas import tpu as pltpu
from jax.experimental.pallas import tpu_sc as plsc

N_NODES = 10000
N_EDGES = 320000
D = 128
NW = 32
EDGES_PER_TILE = N_EDGES // NW
CHUNK = 80
N_CHUNKS = EDGES_PER_TILE // CHUNK
L = 16


def _edge_kernel(z_hbm, src_hbm, dst_hbm, out_hbm,
                 idx_s, idx_d, rows_s, rows_d, out_v, sem_s, sem_d):
    wid = lax.axis_index("s") * 2 + lax.axis_index("c")
    tile_base = wid * EDGES_PER_TILE
    lanes = lax.iota(jnp.int32, L)

    def chunk_body(c, _):
        base = tile_base + c * CHUNK
        pltpu.sync_copy(src_hbm.at[pl.ds(base, CHUNK)], idx_s)
        pltpu.sync_copy(dst_hbm.at[pl.ds(base, CHUNK)], idx_d)
        cp_s = pltpu.async_copy(z_hbm.at[idx_s], rows_s, sem_s)
        cp_d = pltpu.async_copy(z_hbm.at[idx_d], rows_d, sem_d)
        cp_s.wait()
        cp_d.wait()
        for g in range(CHUNK // L):
            erow = lanes + (g * L)
            acc = jnp.zeros((L,), jnp.float32)
            for k in range(D):
                col = jnp.full((L,), k, jnp.int32)
                a = plsc.load_gather(rows_s, [erow, col])
                b = plsc.load_gather(rows_d, [erow, col])
                acc = acc + a * b
            y = 1.0 / (1.0 + jnp.exp(-acc))
            out_v[pl.ds(g * L, L)] = y
        pltpu.sync_copy(out_v, out_hbm.at[pl.ds(base, CHUNK)])
        return _

    lax.fori_loop(0, N_CHUNKS, chunk_body, None)


@jax.jit
def _decode(z, src, dst):
    mesh = plsc.VectorSubcoreMesh(core_axis_name="c", subcore_axis_name="s")
    fn = functools.partial(
        pl.kernel,
        mesh=mesh,
        out_type=jax.ShapeDtypeStruct((N_EDGES,), jnp.float32),
        compiler_params=pltpu.CompilerParams(needs_layout_passes=False),
        scratch_types=[
            pltpu.VMEM((CHUNK,), jnp.int32),
            pltpu.VMEM((CHUNK,), jnp.int32),
            pltpu.VMEM((CHUNK, D), jnp.float32),
            pltpu.VMEM((CHUNK, D), jnp.float32),
            pltpu.VMEM((CHUNK,), jnp.float32),
            pltpu.SemaphoreType.DMA,
            pltpu.SemaphoreType.DMA,
        ],
    )(_edge_kernel)
    return fn(z, src, dst)


def kernel(z, edge_index):
    return _decode(z, edge_index[0], edge_index[1])


# speedup vs baseline: 1.3327x; 1.2157x over previous
import functools

import jax
import jax.numpy as jnp
from jax import lax
from jax.experimental import pallas as pl
from jax.experimental.pallas import tpu as pltpu
from jax.experimental.pallas import tpu_sc as plsc

N_NODES = 10000
N_EDGES = 320000
D = 128
NW = 32
EDGES_PER_TILE = N_EDGES // NW
CHUNK = 80
N_CHUNKS = EDGES_PER_TILE // CHUNK
NBUF = 4
L = 16


def _edge_kernel(z_hbm, src_hbm, dst_hbm, out_hbm,
                 idx_s, idx_d, out_v, rows_s, rows_d, sems_s, sems_d):
    wid = lax.axis_index("s") * 2 + lax.axis_index("c")
    tile_base = wid * EDGES_PER_TILE
    lanes = lax.iota(jnp.int32, L)

    pltpu.sync_copy(src_hbm.at[pl.ds(tile_base, EDGES_PER_TILE)], idx_s)
    pltpu.sync_copy(dst_hbm.at[pl.ds(tile_base, EDGES_PER_TILE)], idx_d)

    def fire(cidx, b):
        off = cidx * CHUNK
        pltpu.async_copy(z_hbm.at[idx_s.at[pl.ds(off, CHUNK)]], rows_s[b],
                         sems_s[b])
        pltpu.async_copy(z_hbm.at[idx_d.at[pl.ds(off, CHUNK)]], rows_d[b],
                         sems_d[b])

    def drain(cidx, b):
        off = cidx * CHUNK
        pltpu.make_async_copy(z_hbm.at[idx_s.at[pl.ds(off, CHUNK)]],
                              rows_s[b], sems_s[b]).wait()
        pltpu.make_async_copy(z_hbm.at[idx_d.at[pl.ds(off, CHUNK)]],
                              rows_d[b], sems_d[b]).wait()

    def compute(cidx, b):
        rs, rd = rows_s[b], rows_d[b]

        def group_body(g, _):
            erow = lanes + g * L
            acc = jnp.zeros((L,), jnp.float32)
            for k in range(D):
                col = jnp.full((L,), k, jnp.int32)
                a = plsc.load_gather(rs, [erow, col])
                bb = plsc.load_gather(rd, [erow, col])
                acc = acc + a * bb
            y = 1.0 / (1.0 + jnp.exp(-acc))
            out_v[pl.ds(cidx * CHUNK + g * L, L)] = y
            return _

        lax.fori_loop(0, CHUNK // L, group_body, None)

    for c in range(NBUF - 1):
        fire(c, c)

    def outer_body(c4, _):
        for b in range(NBUF):
            cidx = c4 * NBUF + b
            nxt = cidx + (NBUF - 1)

            @pl.when(nxt < N_CHUNKS)
            def _():
                fire(nxt, (b + NBUF - 1) % NBUF)

            drain(cidx, b)
            compute(cidx, b)
        return _

    lax.fori_loop(0, (N_CHUNKS - 1) // NBUF, outer_body, None)
    last = N_CHUNKS - 1
    drain(last, last % NBUF)
    compute(last, last % NBUF)

    pltpu.sync_copy(out_v, out_hbm.at[pl.ds(tile_base, EDGES_PER_TILE)])


@jax.jit
def _decode(z, src, dst):
    mesh = plsc.VectorSubcoreMesh(core_axis_name="c", subcore_axis_name="s")
    fn = functools.partial(
        pl.kernel,
        mesh=mesh,
        out_type=jax.ShapeDtypeStruct((N_EDGES,), jnp.float32),
        compiler_params=pltpu.CompilerParams(needs_layout_passes=False),
        scratch_types=[
            pltpu.VMEM((EDGES_PER_TILE,), jnp.int32),
            pltpu.VMEM((EDGES_PER_TILE,), jnp.int32),
            pltpu.VMEM((EDGES_PER_TILE,), jnp.float32),
            [pltpu.VMEM((CHUNK, D), jnp.float32) for _ in range(NBUF)],
            [pltpu.VMEM((CHUNK, D), jnp.float32) for _ in range(NBUF)],
            [pltpu.SemaphoreType.DMA for _ in range(NBUF)],
            [pltpu.SemaphoreType.DMA for _ in range(NBUF)],
        ],
    )(_edge_kernel)
    return fn(z, src, dst)


def kernel(z, edge_index):
    return _decode(z, edge_index[0], edge_index[1])
